# baseline (speedup 1.0000x reference)
import functools

import jax
import jax.numpy as jnp
from jax import lax
from jax.experimental import pallas as pl
from jax.experimental.pallas import tpu as pltpu
from jax.experimental.pallas import tpu_sc as plsc

NUM_COMPONENTS = 64
BATCH = 16384
EPS = 1e-6

NC = 2
NS = 16
NW = NC * NS
B_PER_W = BATCH // NW
NCHUNK = 4
CHUNK = B_PER_W // NCHUNK
NGROUP = CHUNK // 16


def _sqrt16(x):
    i = lax.bitcast_convert_type(x, jnp.int32)
    i = 0x5F3759DF - lax.shift_right_logical(i, 1)
    y = lax.bitcast_convert_type(i, jnp.float32)
    for _ in range(3):
        y = y * (1.5 - 0.5 * x * y * y)
    return x * y


def _sc_body(U_hbm, I_hbm, W_hbm, H_hbm, out_hbm,
             idx_u, idx_i, w_rows, h_rows, out_v, semw, semh):
    wid = lax.axis_index("s") * NC + lax.axis_index("c")
    pltpu.sync_copy(U_hbm.at[wid], idx_u)
    pltpu.sync_copy(I_hbm.at[wid], idx_i)
    cws = []
    chs = []
    for k in range(NCHUNK):
        cws.append(pltpu.async_copy(W_hbm.at[idx_u.at[k]],
                                    w_rows.at[pl.ds(k * CHUNK, CHUNK)],
                                    semw.at[k]))
        chs.append(pltpu.async_copy(H_hbm.at[idx_i.at[k]],
                                    h_rows.at[pl.ds(k * CHUNK, CHUNK)],
                                    semh.at[k]))
    for k in range(NCHUNK):
        cws[k].wait()
        chs[k].wait()

        def group(g, _, k=k):
            rows = (k * CHUNK + g * 16
                    + lax.broadcasted_iota(jnp.int32, (16,), 0))
            acc = jnp.zeros((16,), jnp.float32)
            for c in range(NUM_COMPONENTS):
                cc = jnp.full((16,), c, jnp.int32)
                wv = plsc.load_gather(w_rows, [rows, cc])
                hv = plsc.load_gather(h_rows, [rows, cc])
                d = wv - hv + EPS
                acc = acc + d * d
            out_v[pl.ds(k * CHUNK + g * 16, 16)] = _sqrt16(acc)
            return 0

        lax.fori_loop(0, NGROUP, group, 0)
    pltpu.sync_copy(out_v, out_hbm.at[wid])


@jax.jit
def kernel(U, I, W, H):
    U3 = U.reshape(NW, NCHUNK, CHUNK).astype(jnp.int32)
    I3 = I.reshape(NW, NCHUNK, CHUNK).astype(jnp.int32)
    mesh = plsc.VectorSubcoreMesh(core_axis_name="c", subcore_axis_name="s")
    run = functools.partial(
        pl.kernel,
        mesh=mesh,
        compiler_params=pltpu.CompilerParams(needs_layout_passes=False,
                                             use_tc_tiling_on_sc=False),
        out_type=jax.ShapeDtypeStruct((NW, B_PER_W), jnp.float32),
        scratch_types=[
            pltpu.VMEM((NCHUNK, CHUNK), jnp.int32),
            pltpu.VMEM((NCHUNK, CHUNK), jnp.int32),
            pltpu.VMEM((B_PER_W, NUM_COMPONENTS), jnp.float32),
            pltpu.VMEM((B_PER_W, NUM_COMPONENTS), jnp.float32),
            pltpu.VMEM((B_PER_W,), jnp.float32),
            pltpu.SemaphoreType.DMA((NCHUNK,)),
            pltpu.SemaphoreType.DMA((NCHUNK,)),
        ],
    )(_sc_body)
    out = run(U3, I3, W, H)
    return out.reshape(BATCH)
